# baseline (speedup 1.0000x reference)
import jax
import jax.numpy as jnp
from jax.experimental import pallas as pl
from jax.experimental.pallas import tpu as pltpu

_G = 152
_NA = 3
_NF = 10
_ANCHOR_W = (1.08, 3.42, 6.63)
_ANCHOR_H = (1.19, 4.41, 11.38)


def _decode_block(stride_ref, x_ref, o_ref):
    a = pl.program_id(1)
    s = stride_ref[0, 0]
    aw = jnp.where(a == 0, _ANCHOR_W[0], jnp.where(a == 1, _ANCHOR_W[1], _ANCHOR_W[2]))
    ah = jnp.where(a == 0, _ANCHOR_H[0], jnp.where(a == 1, _ANCHOR_H[1], _ANCHOR_H[2]))
    jj = jax.lax.broadcasted_iota(jnp.int32, (_G, _G), 1).astype(jnp.float32)
    ii = jax.lax.broadcasted_iota(jnp.int32, (_G, _G), 0).astype(jnp.float32)
    sig = jax.nn.sigmoid
    for c in range(_NF):
        v = x_ref[0, 0, c]
        if c == 0:
            r = (sig(v) + jj) * s
        elif c == 1:
            r = (sig(v) + ii) * s
        elif c == 2:
            r = jnp.exp(v) * aw
        elif c == 3:
            r = jnp.exp(v) * ah
        elif c in (4, 5):
            r = v
        else:
            r = sig(v)
        o_ref[c, 0, 0] = r


def kernel(x, img_size):
    n = x.shape[0]
    x5 = x.reshape(n, _NA, _NF, _G, _G)
    stride = (jnp.float32(img_size) / _G).reshape(1, 1)

    out = pl.pallas_call(
        _decode_block,
        grid=(n, _NA),
        in_specs=[
            pl.BlockSpec(memory_space=pltpu.SMEM),
            pl.BlockSpec((1, 1, _NF, _G, _G), lambda b, a: (b, a, 0, 0, 0)),
        ],
        out_specs=pl.BlockSpec((_NF, 1, 1, _G, _G), lambda b, a: (0, b, a, 0, 0)),
        out_shape=jax.ShapeDtypeStruct((_NF, n, _NA, _G, _G), jnp.float32),
    )(stride, x5)
    return jnp.transpose(out, (1, 2, 3, 4, 0)).reshape(n, _NA * _G * _G, _NF)


# speedup vs baseline: 1.1455x; 1.1455x over previous
import jax
import jax.numpy as jnp
from jax.experimental import pallas as pl
from jax.experimental.pallas import tpu as pltpu

_G = 152
_NA = 3
_NF = 10
_ANCHOR_W = (1.08, 3.42, 6.63)
_ANCHOR_H = (1.19, 4.41, 11.38)


def _decode_block(stride_ref, x_ref, o_ref):
    s = stride_ref[0, 0]
    jj = jax.lax.broadcasted_iota(jnp.int32, (_G, _G), 1).astype(jnp.float32)
    ii = jax.lax.broadcasted_iota(jnp.int32, (_G, _G), 0).astype(jnp.float32)
    sig = jax.nn.sigmoid
    for b in range(2):
      for a in range(_NA):
        for c in range(_NF):
            v = x_ref[b, a, c]
            if c == 0:
                r = (sig(v) + jj) * s
            elif c == 1:
                r = (sig(v) + ii) * s
            elif c == 2:
                r = jnp.exp(v) * _ANCHOR_W[a]
            elif c == 3:
                r = jnp.exp(v) * _ANCHOR_H[a]
            elif c in (4, 5):
                r = v
            else:
                r = sig(v)
            o_ref[c, b, a] = r


def kernel(x, img_size):
    n = x.shape[0]
    x5 = x.reshape(n, _NA, _NF, _G, _G)
    stride = (jnp.float32(img_size) / _G).reshape(1, 1)

    out = pl.pallas_call(
        _decode_block,
        grid=(n // 2,),
        in_specs=[
            pl.BlockSpec(memory_space=pltpu.SMEM),
            pl.BlockSpec((2, _NA, _NF, _G, _G), lambda b: (b, 0, 0, 0, 0)),
        ],
        out_specs=pl.BlockSpec((_NF, 2, _NA, _G, _G), lambda b: (0, b, 0, 0, 0)),
        out_shape=jax.ShapeDtypeStruct((_NF, n, _NA, _G, _G), jnp.float32),
    )(stride, x5)
    return jnp.transpose(out, (1, 2, 3, 4, 0)).reshape(n, _NA * _G * _G, _NF)
